# baseline (speedup 1.0000x reference)
import functools

import jax
import jax.numpy as jnp
from jax import lax
from jax.experimental import pallas as pl
from jax.experimental.pallas import tpu as pltpu
from jax.experimental.pallas import tpu_sc as plsc

NUM_CAT = 26
VOCAB = 100000
EMBED = 32
NUM_DENSE = 13
CROSS_LAYERS = 3
BATCH = 4096
EDIM = NUM_CAT * EMBED
DPAD = 128
BB = 128
NW = 32
NQROW = NUM_CAT * VOCAB // 4


@functools.lru_cache(maxsize=1)
def _make_gather():
    info = plsc.get_sparse_core_info()
    nc = info.num_cores
    mesh = plsc.VectorSubcoreMesh(core_axis_name="c", subcore_axis_name="s")

    @functools.partial(
        pl.kernel,
        mesh=mesh,
        out_type=jax.ShapeDtypeStruct((NW, NUM_CAT, BB, 128), jnp.float32),
        scratch_types=[
            pltpu.VMEM((NUM_CAT, 1, BB), jnp.int32),
            pltpu.VMEM((BB, 128), jnp.float32),
            pltpu.VMEM((BB, 128), jnp.float32),
            pltpu.SemaphoreType.DMA,
            pltpu.SemaphoreType.DMA,
        ],
    )
    def gather(tbl_hbm, idx_hbm, out_hbm, idxv, quad0, quad1, gsem, osem):
        wid = lax.axis_index("s") * nc + lax.axis_index("c")
        pltpu.sync_copy(idx_hbm.at[wid], idxv)
        quads = (quad0, quad1)
        gets = {}
        puts = {}
        for f in range(2):
            gets[f] = pltpu.async_copy(
                tbl_hbm.at[idxv.at[f, 0]], quads[f % 2], gsem)
        for f in range(NUM_CAT):
            gets[f].wait()
            puts[f] = pltpu.async_copy(
                quads[f % 2], out_hbm.at[wid, f], osem)
            if f + 2 < NUM_CAT:
                puts[f].wait()
                gets[f + 2] = pltpu.async_copy(
                    tbl_hbm.at[idxv.at[f + 2, 0]], quads[f % 2], gsem)
        puts[NUM_CAT - 2].wait()
        puts[NUM_CAT - 1].wait()

    return gather


def _dense_body(xd_ref, xe_ref, md_ref, cwd_ref, cwe_ref, cbd_ref, cbe_ref,
                w1d_ref, w1e_ref, b1_ref, w2_ref, b2_ref, w3_ref, b3_ref,
                wod_ref, woe_ref, woh_ref, out_ref):
    xd0 = xd_ref[...]
    pieces = []
    for f in range(NUM_CAT):
        q = xe_ref[0, f]
        m = md_ref[0, :, f * EMBED:(f + 1) * EMBED]
        p = jnp.zeros((BB, EMBED), jnp.float32)
        for k in range(4):
            p = p + jnp.where(m == k, q[:, k * EMBED:(k + 1) * EMBED], 0.0)
        pieces.append(p)
    xe0 = jnp.concatenate(pieces, axis=1)
    xd, xe = xd0, xe0
    for i in range(CROSS_LAYERS):
        xw = (jnp.sum(xd * cwd_ref[i:i + 1, :], axis=1, keepdims=True)
              + jnp.sum(xe * cwe_ref[i:i + 1, :], axis=1, keepdims=True))
        xd = xd0 * xw + cbd_ref[i:i + 1, :] + xd
        xe = xe0 * xw + cbe_ref[i:i + 1, :] + xe
    h = jnp.maximum(
        jnp.dot(xd0, w1d_ref[...], preferred_element_type=jnp.float32)
        + jnp.dot(xe0, w1e_ref[...], preferred_element_type=jnp.float32)
        + b1_ref[...], 0.0)
    h = jnp.maximum(
        jnp.dot(h, w2_ref[...], preferred_element_type=jnp.float32)
        + b2_ref[...], 0.0)
    h = jnp.maximum(
        jnp.dot(h, w3_ref[...], preferred_element_type=jnp.float32)
        + b3_ref[...], 0.0)
    out = (jnp.sum(xd * wod_ref[...], axis=1)
           + jnp.sum(xe * woe_ref[...], axis=1)
           + jnp.sum(h * woh_ref[...], axis=1))
    out_ref[0, 0, :] = out


@functools.lru_cache(maxsize=1)
def _make_dense(interpret=False):
    full = lambda i: (0, 0)
    return pl.pallas_call(
        _dense_body,
        grid=(BATCH // BB,),
        in_specs=[
            pl.BlockSpec((BB, DPAD), lambda i: (i, 0)),
            pl.BlockSpec((1, NUM_CAT, BB, 128), lambda i: (i, 0, 0, 0)),
            pl.BlockSpec((1, BB, EDIM), lambda i: (i, 0, 0)),
            pl.BlockSpec((CROSS_LAYERS, DPAD), full),
            pl.BlockSpec((CROSS_LAYERS, EDIM), full),
            pl.BlockSpec((CROSS_LAYERS, DPAD), full),
            pl.BlockSpec((CROSS_LAYERS, EDIM), full),
            pl.BlockSpec((DPAD, 512), full),
            pl.BlockSpec((EDIM, 512), full),
            pl.BlockSpec((1, 512), full),
            pl.BlockSpec((512, 256), full),
            pl.BlockSpec((1, 256), full),
            pl.BlockSpec((256, 128), full),
            pl.BlockSpec((1, 128), full),
            pl.BlockSpec((1, DPAD), full),
            pl.BlockSpec((1, EDIM), full),
            pl.BlockSpec((1, 128), full),
        ],
        out_specs=pl.BlockSpec((1, 1, BB), lambda i: (i, 0, 0)),
        out_shape=jax.ShapeDtypeStruct((BATCH // BB, 1, BB), jnp.float32),
        interpret=interpret,
    )


def kernel(dense, cats, tables, cross_w, cross_b, W1, b1, W2, b2, W3, b3, Wo, bo):
    nd, dp = NUM_DENSE, DPAD - NUM_DENSE
    ci = cats.astype(jnp.int32)
    offs = (jnp.arange(NUM_CAT, dtype=jnp.int32) * VOCAB)[None, :, None]
    flat3 = ci.reshape(NW, BB, NUM_CAT).transpose(0, 2, 1) + offs
    idx4 = (flat3 // 4).reshape(NW, NUM_CAT, 1, BB)
    modm = jnp.repeat(ci % 4, EMBED, axis=1).reshape(NW, BB, EDIM)
    tbl128 = tables.reshape(NQROW, 128)
    emb4 = _make_gather()(tbl128, idx4)
    xd = jnp.pad(dense, ((0, 0), (0, dp)))
    cwd = jnp.pad(cross_w[:, :nd], ((0, 0), (0, dp)))
    cwe = cross_w[:, nd:]
    cbd = jnp.pad(cross_b[:, :nd], ((0, 0), (0, dp)))
    cbe = cross_b[:, nd:]
    w1d = jnp.pad(W1[:nd], ((0, dp), (0, 0)))
    w1e = W1[nd:]
    wod = jnp.pad(Wo[:nd, 0][None, :], ((0, 0), (0, dp)))
    woe = Wo[nd:nd + EDIM, 0][None, :]
    woh = Wo[nd + EDIM:, 0][None, :]
    out3 = _make_dense()(xd, emb4, modm, cwd, cwe, cbd, cbe, w1d, w1e,
                         b1[None, :], W2, b2[None, :], W3, b3[None, :],
                         wod, woe, woh)
    return out3.reshape(BATCH) + bo[0]


# speedup vs baseline: 1.0323x; 1.0323x over previous
import functools

import jax
import jax.numpy as jnp
from jax import lax
from jax.experimental import pallas as pl
from jax.experimental.pallas import tpu as pltpu
from jax.experimental.pallas import tpu_sc as plsc

NUM_CAT = 26
VOCAB = 100000
EMBED = 32
NUM_DENSE = 13
CROSS_LAYERS = 3
BATCH = 4096
EDIM = NUM_CAT * EMBED
DPAD = 128
BB = 128
NW = 32


@functools.lru_cache(maxsize=1)
def _make_gather():
    info = plsc.get_sparse_core_info()
    nc = info.num_cores
    mesh = plsc.VectorSubcoreMesh(core_axis_name="c", subcore_axis_name="s")

    @functools.partial(
        pl.kernel,
        mesh=mesh,
        compiler_params=pltpu.CompilerParams(use_tc_tiling_on_sc=False),
        out_type=jax.ShapeDtypeStruct((NW, NUM_CAT * BB, EMBED), jnp.float32),
        scratch_types=[
            pltpu.VMEM((NUM_CAT, BB), jnp.int32),
            pltpu.VMEM((NUM_CAT * BB, EMBED), jnp.float32),
            pltpu.SemaphoreType.DMA,
        ],
    )
    def gather(tbl_hbm, idx_hbm, out_hbm, idxv, rows, sem):
        wid = lax.axis_index("s") * nc + lax.axis_index("c")
        pltpu.sync_copy(idx_hbm.at[wid], idxv)
        cps = [
            pltpu.async_copy(
                tbl_hbm.at[idxv.at[f]],
                rows.at[pl.ds(f * BB, BB)],
                sem,
            )
            for f in range(NUM_CAT)
        ]
        for c in cps:
            c.wait()
        pltpu.sync_copy(rows, out_hbm.at[wid])

    return gather


def _dense_body(xd_ref, xe_ref, cwd_ref, cwe_ref, cbd_ref, cbe_ref,
                w1d_ref, w1e_ref, b1_ref, w2_ref, b2_ref, w3_ref, b3_ref,
                wod_ref, woe_ref, woh_ref, out_ref):
    xd0 = xd_ref[...]
    xe0 = jnp.concatenate(
        [xe_ref[0, f * BB:(f + 1) * BB, :] for f in range(NUM_CAT)], axis=1)
    xd, xe = xd0, xe0
    for i in range(CROSS_LAYERS):
        xw = (jnp.sum(xd * cwd_ref[i:i + 1, :], axis=1, keepdims=True)
              + jnp.sum(xe * cwe_ref[i:i + 1, :], axis=1, keepdims=True))
        xd = xd0 * xw + cbd_ref[i:i + 1, :] + xd
        xe = xe0 * xw + cbe_ref[i:i + 1, :] + xe
    h = jnp.maximum(
        jnp.dot(xd0, w1d_ref[...], preferred_element_type=jnp.float32)
        + jnp.dot(xe0, w1e_ref[...], preferred_element_type=jnp.float32)
        + b1_ref[...], 0.0)
    h = jnp.maximum(
        jnp.dot(h, w2_ref[...], preferred_element_type=jnp.float32)
        + b2_ref[...], 0.0)
    h = jnp.maximum(
        jnp.dot(h, w3_ref[...], preferred_element_type=jnp.float32)
        + b3_ref[...], 0.0)
    out = (jnp.sum(xd * wod_ref[...], axis=1)
           + jnp.sum(xe * woe_ref[...], axis=1)
           + jnp.sum(h * woh_ref[...], axis=1))
    out_ref[0, 0, :] = out


@functools.lru_cache(maxsize=1)
def _make_dense(interpret=False):
    full = lambda i: (0, 0)
    return pl.pallas_call(
        _dense_body,
        grid=(BATCH // BB,),
        in_specs=[
            pl.BlockSpec((BB, DPAD), lambda i: (i, 0)),
            pl.BlockSpec((1, NUM_CAT * BB, EMBED), lambda i: (i, 0, 0)),
            pl.BlockSpec((CROSS_LAYERS, DPAD), full),
            pl.BlockSpec((CROSS_LAYERS, EDIM), full),
            pl.BlockSpec((CROSS_LAYERS, DPAD), full),
            pl.BlockSpec((CROSS_LAYERS, EDIM), full),
            pl.BlockSpec((DPAD, 512), full),
            pl.BlockSpec((EDIM, 512), full),
            pl.BlockSpec((1, 512), full),
            pl.BlockSpec((512, 256), full),
            pl.BlockSpec((1, 256), full),
            pl.BlockSpec((256, 128), full),
            pl.BlockSpec((1, 128), full),
            pl.BlockSpec((1, DPAD), full),
            pl.BlockSpec((1, EDIM), full),
            pl.BlockSpec((1, 128), full),
        ],
        out_specs=pl.BlockSpec((1, 1, BB), lambda i: (i, 0, 0)),
        out_shape=jax.ShapeDtypeStruct((BATCH // BB, 1, BB), jnp.float32),
        interpret=interpret,
    )


def kernel(dense, cats, tables, cross_w, cross_b, W1, b1, W2, b2, W3, b3, Wo, bo):
    nd, dp = NUM_DENSE, DPAD - NUM_DENSE
    tbl_flat = tables.reshape(NUM_CAT * VOCAB, EMBED)
    offs = (jnp.arange(NUM_CAT, dtype=jnp.int32) * VOCAB)[None, :, None]
    idx3 = (cats.astype(jnp.int32).reshape(NW, BB, NUM_CAT)
            .transpose(0, 2, 1) + offs)
    emb3 = _make_gather()(tbl_flat, idx3)
    xd = jnp.pad(dense, ((0, 0), (0, dp)))
    cwd = jnp.pad(cross_w[:, :nd], ((0, 0), (0, dp)))
    cwe = cross_w[:, nd:]
    cbd = jnp.pad(cross_b[:, :nd], ((0, 0), (0, dp)))
    cbe = cross_b[:, nd:]
    w1d = jnp.pad(W1[:nd], ((0, dp), (0, 0)))
    w1e = W1[nd:]
    wod = jnp.pad(Wo[:nd, 0][None, :], ((0, 0), (0, dp)))
    woe = Wo[nd:nd + EDIM, 0][None, :]
    woh = Wo[nd + EDIM:, 0][None, :]
    out3 = _make_dense()(xd, emb3, cwd, cwe, cbd, cbe, w1d, w1e,
                         b1[None, :], W2, b2[None, :], W3, b3[None, :],
                         wod, woe, woh)
    return out3.reshape(BATCH) + bo[0]
